# baseline (speedup 1.0000x reference)
import functools

import jax
import jax.numpy as jnp
from jax import lax
from jax.experimental import pallas as pl
from jax.experimental.pallas import tpu as pltpu
from jax.experimental.pallas import tpu_sc as plsc

_NUM_CORES = 2
_NUM_SUBCORES = 16
_NUM_WORKERS = _NUM_CORES * _NUM_SUBCORES


@functools.lru_cache(maxsize=None)
def _build(ht, bt, dim):
    assert ht % 8 == 0 and bt % (128 * _NUM_WORKERS) == 0
    stripe = bt // _NUM_WORKERS
    nrows = ht // 8
    nbig = nrows // 2
    assert nrows % 2 == 1 and nbig % 2 == 0 and nbig >= 4
    chunk = 16 * stripe
    tails = 8 * stripe

    slice_sz = (dim // _NUM_SUBCORES) & ~7
    last_sz = dim - (_NUM_SUBCORES - 1) * slice_sz
    bounce = 4096

    mesh = plsc.VectorSubcoreMesh(core_axis_name="c", subcore_axis_name="s")

    @functools.partial(
        pl.kernel,
        mesh=mesh,
        out_type=jax.ShapeDtypeStruct((ht, bt), jnp.float32),
        scratch_types=[
            pltpu.VMEM_SHARED((dim,), jnp.float32),
            pltpu.VMEM((16, 512), jnp.int32),
            pltpu.VMEM((chunk,), jnp.int32),
            pltpu.VMEM((chunk,), jnp.int32),
            pltpu.VMEM((chunk,), jnp.float32),
            pltpu.VMEM((chunk,), jnp.float32),
            pltpu.VMEM((16, 512), jnp.float32),
            pltpu.VMEM((tails,), jnp.int32),
            pltpu.VMEM((tails,), jnp.float32),
            pltpu.VMEM((bounce,), jnp.float32),
            pltpu.VMEM((bounce,), jnp.float32),
            pltpu.SemaphoreType.DMA,
            pltpu.SemaphoreType.DMA,
            pltpu.SemaphoreType.DMA,
            pltpu.SemaphoreType.DMA,
            pltpu.SemaphoreType.DMA,
            pltpu.SemaphoreType.DMA,
        ],
        compiler_params=pltpu.CompilerParams(use_tc_tiling_on_sc=True),
    )
    def gather_kernel(idxT_hbm, w_hbm, outT_hbm, shared,
                      stg_i, iv0, iv1, ov0, ov1, stg_o, idx_vt, out_vt,
                      bn0, bn1, isem, gs0, gs1, osem, ss0, ss1):
        idx_v, out_v = (iv0, iv1), (ov0, ov1)
        gsem, ssem, bnc = (gs0, gs1), (ss0, ss1), (bn0, bn1)
        sid = lax.axis_index("s")
        wid = sid * _NUM_CORES + lax.axis_index("c")
        col0 = wid * stripe
        cols = pl.ds(col0, stripe)

        def mk_a(j):
            return pltpu.make_async_copy(
                idxT_hbm.at[pl.ds(16 * j, 16), cols], stg_i, isem)

        def mk_at():
            return pltpu.make_async_copy(
                idxT_hbm.at[pl.ds(16 * nbig, 8), cols],
                stg_i.at[pl.ds(0, 8)], isem)

        def mk_c(b):
            return pltpu.make_async_copy(shared.at[idx_v[b]], out_v[b], gsem[b])

        def mk_ct():
            return pltpu.make_async_copy(shared.at[idx_vt], out_vt, gsem[0])

        def mk_e(j):
            return pltpu.make_async_copy(
                stg_o, outT_hbm.at[pl.ds(16 * j, 16), cols], osem)

        def mk_et():
            return pltpu.make_async_copy(
                stg_o.at[pl.ds(0, 8)],
                outT_hbm.at[pl.ds(16 * nbig, 8), cols], osem)

        def bridge_in(b):
            for rr in range(16):
                for seg in range(stripe // 16):
                    idx_v[b][pl.ds(rr * stripe + seg * 16, 16)] = (
                        stg_i[rr, pl.ds(seg * 16, 16)])

        def bridge_out(b):
            for rr in range(16):
                for seg in range(stripe // 16):
                    stg_o[rr, pl.ds(seg * 16, 16)] = (
                        out_v[b][pl.ds(rr * stripe + seg * 16, 16)])

        def bridge_in_t():
            for rr in range(8):
                for seg in range(stripe // 16):
                    idx_vt[pl.ds(rr * stripe + seg * 16, 16)] = (
                        stg_i[rr, pl.ds(seg * 16, 16)])

        def bridge_out_t():
            for rr in range(8):
                for seg in range(stripe // 16):
                    stg_o[rr, pl.ds(seg * 16, 16)] = (
                        out_vt[pl.ds(rr * stripe + seg * 16, 16)])

        mk_a(0).start()

        def stage(off, total):
            full, rem = divmod(total, bounce)
            sizes = [bounce] * full + ([rem] if rem else [])
            stores = {}
            for p, sz in enumerate(sizes):
                o = off + p * bounce
                pb = p % 2
                if p >= 2:
                    stores[p - 2].wait()
                pltpu.async_copy(
                    w_hbm.at[pl.ds(o, sz)], bnc[pb].at[pl.ds(0, sz)],
                    ssem[pb]).wait()
                stores[p] = pltpu.async_copy(
                    bnc[pb].at[pl.ds(0, sz)], shared.at[pl.ds(o, sz)], ssem[pb])
            for p in (len(sizes) - 2, len(sizes) - 1):
                if p >= 0:
                    stores[p].wait()

        @pl.when(sid < _NUM_SUBCORES - 1)
        def _():
            stage(sid * slice_sz, slice_sz)

        @pl.when(sid == _NUM_SUBCORES - 1)
        def _():
            stage((_NUM_SUBCORES - 1) * slice_sz, last_sz)

        plsc.subcore_barrier()

        mk_a(0).wait()
        bridge_in(0)
        mk_a(1).start()
        mk_c(0).start()
        mk_a(1).wait()
        bridge_in(1)
        mk_a(2).start()
        mk_c(0).wait()
        mk_c(1).start()
        bridge_out(0)
        mk_e(0).start()

        def sub(j, b):
            mk_a(j).wait()
            bridge_in(b)

            @pl.when(j < nbig - 1)
            def _():
                mk_a(j + 1).start()

            @pl.when(j == nbig - 1)
            def _():
                mk_at().start()

            mk_c(1 - b).wait()
            mk_c(b).start()
            mk_e(j - 2).wait()
            bridge_out(1 - b)
            mk_e(j - 1).start()

        def body(i, carry):
            sub(2 + 2 * i, 0)
            sub(3 + 2 * i, 1)
            return carry

        lax.fori_loop(0, (nbig - 2) // 2, body, None)

        jl = nbig - 1
        mk_at().wait()
        bridge_in_t()
        mk_c(1).wait()
        mk_ct().start()
        mk_e(jl - 1).wait()
        bridge_out(1)
        mk_e(jl).start()
        mk_ct().wait()
        mk_e(jl).wait()
        bridge_out_t()
        mk_et().start()
        mk_et().wait()

    return gather_kernel


def kernel(indexes, weights):
    b, h = indexes.shape
    outT = _build(h, b, weights.shape[0])(indexes.T, weights)
    return outT.T


# speedup vs baseline: 1.1292x; 1.1292x over previous
import functools

import jax
import jax.numpy as jnp
from jax import lax
from jax.experimental import pallas as pl
from jax.experimental.pallas import tpu as pltpu
from jax.experimental.pallas import tpu_sc as plsc

_NUM_CORES = 2
_NUM_SUBCORES = 16
_NUM_WORKERS = _NUM_CORES * _NUM_SUBCORES


@functools.lru_cache(maxsize=None)
def _build(ht, bt, dim):
    assert ht % 8 == 0 and bt % (128 * _NUM_WORKERS) == 0
    nrows = ht // 8
    stripe = bt // _NUM_WORKERS
    chunk = 8 * stripe
    assert nrows % 2 == 1 and nrows >= 5

    slice_sz = (dim // _NUM_SUBCORES) & ~7
    last_sz = dim - (_NUM_SUBCORES - 1) * slice_sz
    bounce = 8192

    mesh = plsc.VectorSubcoreMesh(core_axis_name="c", subcore_axis_name="s")

    @functools.partial(
        pl.kernel,
        mesh=mesh,
        out_type=jax.ShapeDtypeStruct((ht, bt), jnp.float32),
        scratch_types=[
            pltpu.VMEM_SHARED((dim,), jnp.float32),
            pltpu.VMEM((8, 512), jnp.int32),
            pltpu.VMEM((8, 512), jnp.int32),
            pltpu.VMEM((chunk,), jnp.int32),
            pltpu.VMEM((chunk,), jnp.int32),
            pltpu.VMEM((chunk,), jnp.float32),
            pltpu.VMEM((chunk,), jnp.float32),
            pltpu.VMEM((8, 512), jnp.float32),
            pltpu.VMEM((8, 512), jnp.float32),
            pltpu.VMEM((bounce,), jnp.float32),
            pltpu.VMEM((bounce,), jnp.float32),
            pltpu.SemaphoreType.DMA,
            pltpu.SemaphoreType.DMA,
            pltpu.SemaphoreType.DMA,
            pltpu.SemaphoreType.DMA,
            pltpu.SemaphoreType.DMA,
            pltpu.SemaphoreType.DMA,
            pltpu.SemaphoreType.DMA,
            pltpu.SemaphoreType.DMA,
        ],
        compiler_params=pltpu.CompilerParams(use_tc_tiling_on_sc=True),
    )
    def gather_kernel(idxT_hbm, w_hbm, outT_hbm, shared,
                      si0, si1, iv0, iv1, ov0, ov1, so0, so1, bn0, bn1,
                      is0, is1, gs0, gs1, os0, os1, ss0, ss1):
        stg_i, idx_v, out_v, stg_o = (si0, si1), (iv0, iv1), (ov0, ov1), (so0, so1)
        isem, gsem, osem, ssem = (is0, is1), (gs0, gs1), (os0, os1), (ss0, ss1)
        bnc = (bn0, bn1)
        sid = lax.axis_index("s")
        wid = sid * _NUM_CORES + lax.axis_index("c")
        col0 = wid * stripe

        def rows(j):
            return pl.ds(8 * j, 8)

        def mk_a(j, b):
            return pltpu.make_async_copy(
                idxT_hbm.at[rows(j), pl.ds(col0, stripe)], stg_i[b], isem[b])

        def mk_c(b):
            return pltpu.make_async_copy(shared.at[idx_v[b]], out_v[b], gsem[b])

        def mk_e(j, b):
            return pltpu.make_async_copy(
                stg_o[b], outT_hbm.at[rows(j), pl.ds(col0, stripe)], osem[b])

        def bridge_in(b):
            for rr in range(8):
                for seg in range(stripe // 16):
                    idx_v[b][pl.ds(rr * stripe + seg * 16, 16)] = (
                        stg_i[b][rr, pl.ds(seg * 16, 16)])

        def bridge_out(b):
            for rr in range(8):
                for seg in range(stripe // 16):
                    stg_o[b][rr, pl.ds(seg * 16, 16)] = (
                        out_v[b][pl.ds(rr * stripe + seg * 16, 16)])

        mk_a(0, 0).start()
        mk_a(1, 1).start()

        def stage(off, total):
            full, rem = divmod(total, bounce)
            sizes = [bounce] * full + ([rem] if rem else [])
            loads, stores = {}, {}
            for p, sz in enumerate(sizes):
                o = off + p * bounce
                pb = p % 2
                if p >= 2:
                    stores[p - 2].wait()
                loads[p] = pltpu.async_copy(
                    w_hbm.at[pl.ds(o, sz)], bnc[pb].at[pl.ds(0, sz)], ssem[pb])
                loads[p].wait()
                stores[p] = pltpu.async_copy(
                    bnc[pb].at[pl.ds(0, sz)], shared.at[pl.ds(o, sz)], ssem[pb])
            for p in (len(sizes) - 2, len(sizes) - 1):
                if p >= 0:
                    stores[p].wait()

        @pl.when(sid < _NUM_SUBCORES - 1)
        def _():
            stage(sid * slice_sz, slice_sz)

        @pl.when(sid == _NUM_SUBCORES - 1)
        def _():
            stage((_NUM_SUBCORES - 1) * slice_sz, last_sz)

        plsc.subcore_barrier()

        def sub(i, j, b):
            mk_a(j, b).wait()
            bridge_in(b)

            @pl.when(j >= 1)
            def _():
                mk_c(1 - b).wait()

            mk_c(b).start()

            @pl.when(j >= 1)
            def _():
                @pl.when(j >= 3)
                def _():
                    mk_e(j - 3, 1 - b).wait()

                bridge_out(1 - b)
                mk_e(j - 1, 1 - b).start()

            @pl.when(j + 2 <= nrows - 1)
            def _():
                mk_a(j + 2, b).start()

        def body(i, carry):
            sub(i, 2 * i, 0)
            sub(i, 2 * i + 1, 1)
            return carry

        lax.fori_loop(0, nrows // 2, body, None)

        jl = nrows - 1
        mk_a(jl, 0).wait()
        bridge_in(0)
        mk_c(1).wait()
        mk_c(0).start()
        mk_e(jl - 3, 1).wait()
        bridge_out(1)
        mk_e(jl - 1, 1).start()
        mk_c(0).wait()
        mk_e(jl - 2, 0).wait()
        bridge_out(0)
        mk_e(jl, 0).start()
        mk_e(jl - 1, 1).wait()
        mk_e(jl, 0).wait()

    return gather_kernel


def kernel(indexes, weights):
    b, h = indexes.shape
    outT = _build(h, b, weights.shape[0])(indexes.T, weights)
    return outT.T


# speedup vs baseline: 1.1710x; 1.0370x over previous
import functools

import jax
import jax.numpy as jnp
from jax import lax
from jax.experimental import pallas as pl
from jax.experimental.pallas import tpu as pltpu
from jax.experimental.pallas import tpu_sc as plsc

_NUM_CORES = 2
_NUM_SUBCORES = 16
_NUM_WORKERS = _NUM_CORES * _NUM_SUBCORES


@functools.lru_cache(maxsize=None)
def _build(ht, bt, dim):
    assert ht % 8 == 0 and bt % (128 * _NUM_WORKERS) == 0
    nrows = ht // 8
    stripe = bt // _NUM_WORKERS
    chunk = 8 * stripe
    assert nrows % 2 == 1 and nrows >= 5

    slice_sz = (dim // _NUM_SUBCORES) & ~7
    last_sz = dim - (_NUM_SUBCORES - 1) * slice_sz
    bounce = 16384

    mesh = plsc.VectorSubcoreMesh(core_axis_name="c", subcore_axis_name="s")

    @functools.partial(
        pl.kernel,
        mesh=mesh,
        out_type=jax.ShapeDtypeStruct((ht, bt), jnp.float32),
        scratch_types=[
            pltpu.VMEM_SHARED((dim,), jnp.float32),
            pltpu.VMEM((8, 512), jnp.int32),
            pltpu.VMEM((8, 512), jnp.int32),
            pltpu.VMEM((chunk,), jnp.int32),
            pltpu.VMEM((chunk,), jnp.int32),
            pltpu.VMEM((chunk,), jnp.float32),
            pltpu.VMEM((chunk,), jnp.float32),
            pltpu.VMEM((8, 512), jnp.float32),
            pltpu.VMEM((8, 512), jnp.float32),
            pltpu.VMEM((bounce,), jnp.float32),
            pltpu.VMEM((bounce,), jnp.float32),
            pltpu.SemaphoreType.DMA,
            pltpu.SemaphoreType.DMA,
            pltpu.SemaphoreType.DMA,
            pltpu.SemaphoreType.DMA,
            pltpu.SemaphoreType.DMA,
            pltpu.SemaphoreType.DMA,
            pltpu.SemaphoreType.DMA,
            pltpu.SemaphoreType.DMA,
        ],
        compiler_params=pltpu.CompilerParams(use_tc_tiling_on_sc=True),
    )
    def gather_kernel(idxT_hbm, w_hbm, outT_hbm, shared,
                      si0, si1, iv0, iv1, ov0, ov1, so0, so1, bn0, bn1,
                      is0, is1, gs0, gs1, os0, os1, ss0, ss1):
        stg_i, idx_v, out_v, stg_o = (si0, si1), (iv0, iv1), (ov0, ov1), (so0, so1)
        isem, gsem, osem, ssem = (is0, is1), (gs0, gs1), (os0, os1), (ss0, ss1)
        bnc = (bn0, bn1)
        sid = lax.axis_index("s")
        wid = sid * _NUM_CORES + lax.axis_index("c")
        col0 = wid * stripe

        def rows(j):
            return pl.ds(8 * j, 8)

        def mk_a(j, b):
            return pltpu.make_async_copy(
                idxT_hbm.at[rows(j), pl.ds(col0, stripe)], stg_i[b], isem[b])

        def mk_c(b):
            return pltpu.make_async_copy(shared.at[idx_v[b]], out_v[b], gsem[b])

        def mk_e(j, b):
            return pltpu.make_async_copy(
                stg_o[b], outT_hbm.at[rows(j), pl.ds(col0, stripe)], osem[b])

        def bridge_in(b):
            for rr in range(8):
                for seg in range(stripe // 16):
                    idx_v[b][pl.ds(rr * stripe + seg * 16, 16)] = (
                        stg_i[b][rr, pl.ds(seg * 16, 16)])

        def bridge_out(b):
            for rr in range(8):
                for seg in range(stripe // 16):
                    stg_o[b][rr, pl.ds(seg * 16, 16)] = (
                        out_v[b][pl.ds(rr * stripe + seg * 16, 16)])

        mk_a(0, 0).start()
        mk_a(1, 1).start()

        def stage(off, total):
            full, rem = divmod(total, bounce)
            sizes = [bounce] * full + ([rem] if rem else [])
            loads, stores = {}, {}
            for p, sz in enumerate(sizes):
                o = off + p * bounce
                pb = p % 2
                if p >= 2:
                    stores[p - 2].wait()
                loads[p] = pltpu.async_copy(
                    w_hbm.at[pl.ds(o, sz)], bnc[pb].at[pl.ds(0, sz)], ssem[pb])
                loads[p].wait()
                stores[p] = pltpu.async_copy(
                    bnc[pb].at[pl.ds(0, sz)], shared.at[pl.ds(o, sz)], ssem[pb])
            for p in (len(sizes) - 2, len(sizes) - 1):
                if p >= 0:
                    stores[p].wait()

        @pl.when(sid < _NUM_SUBCORES - 1)
        def _():
            stage(sid * slice_sz, slice_sz)

        @pl.when(sid == _NUM_SUBCORES - 1)
        def _():
            stage((_NUM_SUBCORES - 1) * slice_sz, last_sz)

        plsc.subcore_barrier()

        def sub(i, j, b):
            mk_a(j, b).wait()
            bridge_in(b)

            @pl.when(j >= 1)
            def _():
                mk_c(1 - b).wait()

            mk_c(b).start()

            @pl.when(j >= 1)
            def _():
                @pl.when(j >= 3)
                def _():
                    mk_e(j - 3, 1 - b).wait()

                bridge_out(1 - b)
                mk_e(j - 1, 1 - b).start()

            @pl.when(j + 2 <= nrows - 1)
            def _():
                mk_a(j + 2, b).start()

        def body(i, carry):
            sub(i, 2 * i, 0)
            sub(i, 2 * i + 1, 1)
            return carry

        lax.fori_loop(0, nrows // 2, body, None)

        jl = nrows - 1
        mk_a(jl, 0).wait()
        bridge_in(0)
        mk_c(1).wait()
        mk_c(0).start()
        mk_e(jl - 3, 1).wait()
        bridge_out(1)
        mk_e(jl - 1, 1).start()
        mk_c(0).wait()
        mk_e(jl - 2, 0).wait()
        bridge_out(0)
        mk_e(jl, 0).start()
        mk_e(jl - 1, 1).wait()
        mk_e(jl, 0).wait()

    return gather_kernel


def kernel(indexes, weights):
    b, h = indexes.shape
    outT = _build(h, b, weights.shape[0])(indexes.T, weights)
    return outT.T
